# baseline (speedup 1.0000x reference)
import jax
import jax.numpy as jnp
from jax import lax
from jax.experimental import pallas as pl
from jax.experimental.pallas import tpu as pltpu
from jax.experimental.pallas import tpu_sc as plsc

B = 1024
V = 6890
F = 13776
J = 32
NC = 1
NS = 16
NW = NC * NS
PPW = 6
AUXF = 128
WMAX = AUXF + F * 3 - 32

CP = pltpu.CompilerParams(use_tc_tiling_on_sc=True, needs_layout_passes=False)


def _body(vT_hbm, aux_hbm, out_hbm, reg_v, fwin_v, vbuf_v, obuf_v, semf, semv):
    wid = lax.axis_index("s") * NC + lax.axis_index("c")
    lane = lax.iota(jnp.int32, 16)

    pltpu.sync_copy(aux_hbm.at[pl.ds(0, 128)], reg_v)
    f0 = reg_v[pl.ds(0, 16)]
    f1 = reg_v[pl.ds(16, 16)]
    b_vecs = [plsc.bitcast(reg_v[pl.ds(32 + 16 * i, 16)], jnp.float32)
              for i in range(6)]

    ps = [PPW * wid + e for e in range(PPW)]

    woffs = []
    fcopies = []
    for e in range(PPW):
        j = ps[e] % 32
        fsel = jnp.where(j < 16, f0, f1)
        fid_j = jnp.sum(jnp.where(lane == j % 16, fsel, 0))
        w = AUXF + fid_j * 3
        a = pl.multiple_of(jnp.minimum((w // 16) * 16, WMAX), 16)
        woffs.append(w - a)
        fcopies.append(
            pltpu.async_copy(aux_hbm.at[pl.ds(a, 32)],
                             fwin_v.at[pl.ds(e * 32, 32)], semf))

    vcopies = []
    wgts = []
    for e in range(PPW):
        fcopies[e].wait()
        p = ps[e]
        c = p // 32
        j = p % 32
        lo = fwin_v[pl.ds(e * 32, 16)]
        hi = fwin_v[pl.ds(e * 32 + 16, 16)]
        for k in range(3):
            t = woffs[e] + k
            vsel = jnp.where(t < 16, lo, hi)
            vid = jnp.sum(jnp.where(lane == t % 16, vsel, 0))
            vcopies.append(
                pltpu.async_copy(vT_hbm.at[c, vid],
                                 vbuf_v.at[pl.ds((e * 3 + k) * B, B)], semv))
            tt = j * 3 + k
            vi = tt // 16
            bsel = b_vecs[5]
            for n in range(4, -1, -1):
                bsel = jnp.where(vi == n, b_vecs[n], bsel)
            wgts.append(jnp.sum(jnp.where(lane == tt % 16, bsel, 0.0)))

    for cpy in vcopies:
        cpy.wait()

    for e in range(PPW):
        w0, w1, w2 = wgts[3 * e], wgts[3 * e + 1], wgts[3 * e + 2]

        def comp(m, carry, e=e, w0=w0, w1=w1, w2=w2):
            off = pl.multiple_of(m * 64, 64)
            for u in range(4):
                acc = (w0 * vbuf_v[pl.ds((3 * e) * B + off + u * 16, 16)]
                       + w1 * vbuf_v[pl.ds((3 * e + 1) * B + off + u * 16, 16)]
                       + w2 * vbuf_v[pl.ds((3 * e + 2) * B + off + u * 16, 16)])
                obuf_v[pl.ds(e * B + off + u * 16, 16)] = acc
            return carry

        lax.fori_loop(0, B // 64, comp, 0)

    for e in range(PPW):
        p = ps[e]
        pltpu.sync_copy(obuf_v.at[pl.ds(e * B, B)],
                        out_hbm.at[p // 32, p % 32])


@jax.jit
def _joints_sc(vT, aux):
    mesh = plsc.VectorSubcoreMesh(core_axis_name="c", subcore_axis_name="s", num_cores=1)
    fn = pl.kernel(
        _body,
        out_type=jax.ShapeDtypeStruct((3, J, B), jnp.float32),
        mesh=mesh,
        scratch_types=[
            pltpu.VMEM((128,), jnp.int32),
            pltpu.VMEM((PPW * 32,), jnp.int32),
            pltpu.VMEM((3 * PPW * B,), jnp.float32),
            pltpu.VMEM((PPW * B,), jnp.float32),
            pltpu.SemaphoreType.DMA,
            pltpu.SemaphoreType.DMA,
        ],
        compiler_params=CP,
    )
    return fn(vT, aux)


def kernel(vertices, faces, face_ids, bcs):
    vT = jnp.transpose(vertices, (2, 1, 0))
    aux = jnp.concatenate([
        face_ids.astype(jnp.int32),
        jax.lax.bitcast_convert_type(bcs.reshape(J * 3), jnp.int32),
        faces.reshape(F * 3),
    ])
    out_t = _joints_sc(vT, aux)
    return jnp.transpose(out_t, (2, 1, 0))


# speedup vs baseline: 1.5206x; 1.5206x over previous
import jax
import jax.numpy as jnp
from jax import lax
from jax.experimental import pallas as pl
from jax.experimental.pallas import tpu as pltpu
from jax.experimental.pallas import tpu_sc as plsc

B = 1024
V = 6890
F = 13776
J = 32
NC = 2
NS = 16
NW = NC * NS
PPW = 3

CP = pltpu.CompilerParams(use_tc_tiling_on_sc=True, needs_layout_passes=False)


def _body(vT_hbm, fT_hbm, aux_hbm, out_hbm, reg_v, fwin_v, vbuf_v, obuf_v,
          semf, semv, semw):
    wid = lax.axis_index("s") * NC + lax.axis_index("c")
    lane = lax.iota(jnp.int32, 16)

    pltpu.sync_copy(aux_hbm, reg_v)
    f0 = reg_v[pl.ds(0, 16)]
    f1 = reg_v[pl.ds(16, 16)]
    b_vecs = [plsc.bitcast(reg_v[pl.ds(32 + 16 * i, 16)], jnp.float32)
              for i in range(6)]

    ps = [3 * wid + e for e in range(PPW)]

    fcols = []
    fcopies = []
    for e in range(PPW):
        j = ps[e] % 32
        fsel = jnp.where(j < 16, f0, f1)
        fid_j = jnp.sum(jnp.where(lane == j % 16, fsel, 0))
        a = pl.multiple_of((fid_j // 16) * 16, 16)
        fcols.append(fid_j % 16)
        for k in range(3):
            fcopies.append(
                pltpu.async_copy(fT_hbm.at[k, pl.ds(a, 16)],
                                 fwin_v.at[pl.ds((e * 3 + k) * 16, 16)], semf))

    vcopies = []
    wgts = []
    for e in range(PPW):
        p = ps[e]
        c = p // 32
        j = p % 32
        for k in range(3):
            fcopies[3 * e + k].wait()
            win = fwin_v[pl.ds((e * 3 + k) * 16, 16)]
            vid = jnp.sum(jnp.where(lane == fcols[e], win, 0))
            vcopies.append(
                pltpu.async_copy(vT_hbm.at[c, vid],
                                 vbuf_v.at[pl.ds((e * 3 + k) * B, B)], semv))
            tt = j * 3 + k
            vi = tt // 16
            bsel = b_vecs[5]
            for n in range(4, -1, -1):
                bsel = jnp.where(vi == n, b_vecs[n], bsel)
            wgts.append(jnp.sum(jnp.where(lane == tt % 16, bsel, 0.0)))

    wcopies = []
    for e in range(PPW):
        for k in range(3):
            vcopies[3 * e + k].wait()
        w0, w1, w2 = wgts[3 * e], wgts[3 * e + 1], wgts[3 * e + 2]

        def comp(m, carry, e=e, w0=w0, w1=w1, w2=w2):
            off = pl.multiple_of(m * 128, 128)
            for u in range(8):
                o16 = off + u * 16
                acc = (w0 * vbuf_v[pl.ds((3 * e) * B + o16, 16)]
                       + w1 * vbuf_v[pl.ds((3 * e + 1) * B + o16, 16)]
                       + w2 * vbuf_v[pl.ds((3 * e + 2) * B + o16, 16)])
                obuf_v[pl.ds(e * B + o16, 16)] = acc
            return carry

        lax.fori_loop(0, B // 128, comp, 0)
        wcopies.append(
            pltpu.async_copy(obuf_v.at[pl.ds(e * B, B)],
                             out_hbm.at[ps[e] // 32, ps[e] % 32], semw))

    for cpy in wcopies:
        cpy.wait()


@jax.jit
def _joints_sc(vT, fT, aux):
    mesh = plsc.VectorSubcoreMesh(core_axis_name="c", subcore_axis_name="s")
    fn = pl.kernel(
        _body,
        out_type=jax.ShapeDtypeStruct((3, J, B), jnp.float32),
        mesh=mesh,
        scratch_types=[
            pltpu.VMEM((128,), jnp.int32),
            pltpu.VMEM((PPW * 3 * 16,), jnp.int32),
            pltpu.VMEM((3 * PPW * B,), jnp.float32),
            pltpu.VMEM((PPW * B,), jnp.float32),
            pltpu.SemaphoreType.DMA,
            pltpu.SemaphoreType.DMA,
            pltpu.SemaphoreType.DMA,
        ],
        compiler_params=CP,
    )
    return fn(vT, fT, aux)


def kernel(vertices, faces, face_ids, bcs):
    vT = jnp.transpose(vertices, (2, 1, 0))
    fT = jnp.transpose(faces, (1, 0))
    aux = jnp.concatenate([
        face_ids.astype(jnp.int32),
        jax.lax.bitcast_convert_type(bcs.reshape(J * 3), jnp.int32),
    ])
    out_t = _joints_sc(vT, fT, aux)
    return jnp.transpose(out_t, (2, 1, 0))
